# baseline (speedup 1.0000x reference)
import functools

import jax
import jax.numpy as jnp
from jax import lax
from jax.experimental import pallas as pl
from jax.experimental.pallas import tpu as pltpu
from jax.experimental.pallas import tpu_sc as plsc

N, A, B, C = 128, 64, 64, 64
BATCH = 4096
TOTAL = N * BATCH
LANES = 16

_info = plsc.get_sparse_core_info()
NC = _info.num_cores
NS = _info.num_subcores
NW = NC * NS
PER_W = TOTAL // NW
VEC_PER_ROW = BATCH // LANES
UNROLL = 8
ROWS_PER_W = N // NW


def _sc_body(table, a_h, b_h, c_h, out, a_v, b_v, c_v, idx_v, res_v, sem):
    wid = lax.axis_index("s") * NC + lax.axis_index("c")
    base = wid * PER_W

    pltpu.sync_copy(a_h.at[pl.ds(base, PER_W)], a_v)
    pltpu.sync_copy(b_h.at[pl.ds(base, PER_W)], b_v)
    pltpu.sync_copy(c_h.at[pl.ds(base, PER_W)], c_v)

    row0 = wid * ROWS_PER_W

    def idx_body(j, _):
        row = row0 + j // (VEC_PER_ROW // UNROLL)
        hi = jnp.full((LANES,), row << 18, jnp.int32)
        for u in range(UNROLL):
            s = pl.ds((j * UNROLL + u) * LANES, LANES)
            idx_v[s] = hi | (a_v[s] << 12) | (b_v[s] << 6) | c_v[s]
        return 0

    lax.fori_loop(0, PER_W // LANES // UNROLL, idx_body, 0)

    pltpu.async_copy(table.at[idx_v], res_v, sem).wait()

    pltpu.sync_copy(res_v, out.at[pl.ds(base, PER_W)])


@jax.jit
def _sc_gather(table, a_f, b_f, c_f):
    mesh = plsc.VectorSubcoreMesh(core_axis_name="c", subcore_axis_name="s")
    return pl.kernel(
        _sc_body,
        mesh=mesh,
        out_type=jax.ShapeDtypeStruct((TOTAL,), jnp.float32),
        scratch_types=[
            pltpu.VMEM((PER_W,), jnp.int32),
            pltpu.VMEM((PER_W,), jnp.int32),
            pltpu.VMEM((PER_W,), jnp.int32),
            pltpu.VMEM((PER_W,), jnp.int32),
            pltpu.VMEM((PER_W,), jnp.float32),
            pltpu.SemaphoreType.DMA,
        ],
    )(table, a_f, b_f, c_f)


def kernel(joint, a, b, c):
    table = joint.reshape(-1)
    a_f = a.reshape(-1).astype(jnp.int32)
    b_f = b.reshape(-1).astype(jnp.int32)
    c_f = c.reshape(-1).astype(jnp.int32)
    out = _sc_gather(table, a_f, b_f, c_f)
    return out.reshape(N, BATCH)


# speedup vs baseline: 5.8570x; 5.8570x over previous
import functools

import jax
import jax.numpy as jnp
from jax import lax
from jax.experimental import pallas as pl
from jax.experimental.pallas import tpu as pltpu
from jax.experimental.pallas import tpu_sc as plsc

N, A, B, C = 128, 64, 64, 64
BATCH = 4096
TABLE_WORDS = N * A * B * C
LANES = 16

_info = plsc.get_sparse_core_info()
NC = _info.num_cores
NS = _info.num_subcores
NW = NC * NS
ROWS_W = 8
COLS_W = 2048
PER_W = ROWS_W * COLS_W
UNROLL = 8


def _sc_body(table_h, a_h, b_h, c_h, out_h, a_v, b_v, c_v, idx_v, res_v, sem):
    wid = lax.axis_index("s") * NC + lax.axis_index("c")
    g = wid >> 1
    col0 = (wid & 1) * COLS_W

    for r in range(ROWS_W):
        pltpu.sync_copy(a_h.at[ROWS_W * g + r, pl.ds(col0, COLS_W)],
                        a_v.at[pl.ds(r * COLS_W, COLS_W)])
        pltpu.sync_copy(b_h.at[ROWS_W * g + r, pl.ds(col0, COLS_W)],
                        b_v.at[pl.ds(r * COLS_W, COLS_W)])
        pltpu.sync_copy(c_h.at[ROWS_W * g + r, pl.ds(col0, COLS_W)],
                        c_v.at[pl.ds(r * COLS_W, COLS_W)])

    n_base = ROWS_W * g

    def idx_body(j, _):
        n_vec = jnp.full((LANES,), n_base + (j >> 4), jnp.int32)
        for u in range(UNROLL):
            s = pl.ds((j * UNROLL + u) * LANES, LANES)
            idx_v[s] = (a_v[s] << 19) | (b_v[s] << 13) | (c_v[s] << 7) | n_vec
        return 0

    lax.fori_loop(0, PER_W // LANES // UNROLL, idx_body, 0)

    pltpu.async_copy(table_h.at[idx_v], res_v, sem).wait()

    for r in range(ROWS_W):
        pltpu.sync_copy(res_v.at[pl.ds(r * COLS_W, COLS_W)],
                        out_h.at[ROWS_W * g + r, pl.ds(col0, COLS_W)])


@jax.jit
def _sc_gather(table, a, b, c):
    mesh = plsc.VectorSubcoreMesh(core_axis_name="c", subcore_axis_name="s")
    return pl.kernel(
        _sc_body,
        mesh=mesh,
        out_type=jax.ShapeDtypeStruct((N, BATCH), jnp.float32),
        scratch_types=[
            pltpu.VMEM((PER_W,), jnp.int32),
            pltpu.VMEM((PER_W,), jnp.int32),
            pltpu.VMEM((PER_W,), jnp.int32),
            pltpu.VMEM((PER_W,), jnp.int32),
            pltpu.VMEM((PER_W,), jnp.float32),
            pltpu.SemaphoreType.DMA,
        ],
    )(table, a, b, c)


def kernel(joint, a, b, c):
    table = jnp.transpose(joint, (1, 2, 3, 0)).reshape(-1)
    return _sc_gather(
        table,
        a.astype(jnp.int32),
        b.astype(jnp.int32),
        c.astype(jnp.int32),
    )


# speedup vs baseline: 8.3089x; 1.4186x over previous
import functools

import jax
import jax.numpy as jnp
from jax import lax
from jax.experimental import pallas as pl
from jax.experimental.pallas import tpu as pltpu
from jax.experimental.pallas import tpu_sc as plsc

N, A, B, C = 128, 64, 64, 64
BATCH = 4096
TABLE_WORDS = N * A * B * C
LANES = 16

_info = plsc.get_sparse_core_info()
NC = _info.num_cores
NS = _info.num_subcores
NW = NC * NS
ROWS_W = 8
COLS_W = 2048
PER_W = ROWS_W * COLS_W
UNROLL = 8


def _sc_body(table_h, a_h, b_h, c_h, out_h, a_v, b_v, c_v, idx_v, res_v,
             sem, sem2):
    wid = lax.axis_index("s") * NC + lax.axis_index("c")
    g = wid >> 1
    col0 = (wid & 1) * COLS_W
    n_base = ROWS_W * g

    for r in range(ROWS_W):
        pltpu.async_copy(a_h.at[n_base + r, pl.ds(col0, COLS_W)],
                         a_v.at[pl.ds(r * COLS_W, COLS_W)], sem2)
        pltpu.async_copy(b_h.at[n_base + r, pl.ds(col0, COLS_W)],
                         b_v.at[pl.ds(r * COLS_W, COLS_W)], sem2)
        pltpu.async_copy(c_h.at[n_base + r, pl.ds(col0, COLS_W)],
                         c_v.at[pl.ds(r * COLS_W, COLS_W)], sem2)
    for r in range(ROWS_W):
        for v in (a_v, b_v, c_v):
            pltpu.make_async_copy(
                a_h.at[n_base, pl.ds(col0, COLS_W)],
                v.at[pl.ds(r * COLS_W, COLS_W)], sem2).wait()

    for r in range(ROWS_W):
        def idx_body(j, _, r=r):
            n_vec = jnp.full((LANES,), n_base + r, jnp.int32)
            for u in range(UNROLL):
                s = pl.ds(r * COLS_W + (j * UNROLL + u) * LANES, LANES)
                idx_v[s] = (a_v[s] << 19) | (b_v[s] << 13) | (c_v[s] << 7) | n_vec
            return 0

        lax.fori_loop(0, COLS_W // LANES // UNROLL, idx_body, 0)
        s_row = pl.ds(r * COLS_W, COLS_W)
        pltpu.async_copy(table_h.at[idx_v.at[s_row]], res_v.at[s_row], sem)

    for r in range(ROWS_W):
        s_row = pl.ds(r * COLS_W, COLS_W)
        pltpu.make_async_copy(table_h.at[idx_v.at[s_row]],
                              res_v.at[s_row], sem).wait()

    for r in range(ROWS_W):
        pltpu.async_copy(res_v.at[pl.ds(r * COLS_W, COLS_W)],
                         out_h.at[n_base + r, pl.ds(col0, COLS_W)], sem2)
    for r in range(ROWS_W):
        pltpu.make_async_copy(res_v.at[pl.ds(r * COLS_W, COLS_W)],
                              out_h.at[n_base + r, pl.ds(col0, COLS_W)],
                              sem2).wait()


@jax.jit
def _sc_gather(table, a, b, c):
    mesh = plsc.VectorSubcoreMesh(core_axis_name="c", subcore_axis_name="s")
    return pl.kernel(
        _sc_body,
        mesh=mesh,
        out_type=jax.ShapeDtypeStruct((N, BATCH), jnp.float32),
        scratch_types=[
            pltpu.VMEM((PER_W,), jnp.int32),
            pltpu.VMEM((PER_W,), jnp.int32),
            pltpu.VMEM((PER_W,), jnp.int32),
            pltpu.VMEM((PER_W,), jnp.int32),
            pltpu.VMEM((PER_W,), jnp.float32),
            pltpu.SemaphoreType.DMA,
            pltpu.SemaphoreType.DMA,
        ],
    )(table, a, b, c)


def kernel(joint, a, b, c):
    table = jnp.transpose(joint, (1, 2, 3, 0)).reshape(-1)
    return _sc_gather(
        table,
        a.astype(jnp.int32),
        b.astype(jnp.int32),
        c.astype(jnp.int32),
    )


# speedup vs baseline: 8.5616x; 1.0304x over previous
import functools

import jax
import jax.numpy as jnp
from jax import lax
from jax.experimental import pallas as pl
from jax.experimental.pallas import tpu as pltpu
from jax.experimental.pallas import tpu_sc as plsc

N, A, B, C = 128, 64, 64, 64
BATCH = 4096
TABLE_WORDS = N * A * B * C
LANES = 16

_info = plsc.get_sparse_core_info()
NC = _info.num_cores
NS = _info.num_subcores
NW = NC * NS
ROWS_W = 8
COLS_W = 2048
PER_W = ROWS_W * COLS_W
UNROLL = 8


def _sc_body(table_h, a_h, b_h, c_h, out_h, a_v, b_v, c_v, idx_v, res_v,
             sems, sem2):
    wid = lax.axis_index("s") * NC + lax.axis_index("c")
    g = wid >> 1
    col0 = (wid & 1) * COLS_W
    n_base = ROWS_W * g

    for r in range(ROWS_W):
        pltpu.async_copy(a_h.at[n_base + r, pl.ds(col0, COLS_W)],
                         a_v.at[pl.ds(r * COLS_W, COLS_W)], sem2)
        pltpu.async_copy(b_h.at[n_base + r, pl.ds(col0, COLS_W)],
                         b_v.at[pl.ds(r * COLS_W, COLS_W)], sem2)
        pltpu.async_copy(c_h.at[n_base + r, pl.ds(col0, COLS_W)],
                         c_v.at[pl.ds(r * COLS_W, COLS_W)], sem2)

    for r in range(ROWS_W):
        for v in (a_v, b_v, c_v):
            pltpu.make_async_copy(
                a_h.at[n_base, pl.ds(col0, COLS_W)],
                v.at[pl.ds(r * COLS_W, COLS_W)], sem2).wait()

        def idx_body(j, _, r=r):
            n_vec = jnp.full((LANES,), n_base + r, jnp.int32)
            for u in range(UNROLL):
                s = pl.ds(r * COLS_W + (j * UNROLL + u) * LANES, LANES)
                idx_v[s] = (a_v[s] << 19) | (b_v[s] << 13) | (c_v[s] << 7) | n_vec
            return 0

        lax.fori_loop(0, COLS_W // LANES // UNROLL, idx_body, 0)
        s_row = pl.ds(r * COLS_W, COLS_W)
        pltpu.async_copy(table_h.at[idx_v.at[s_row]], res_v.at[s_row],
                         sems.at[r])

    for r in range(ROWS_W):
        s_row = pl.ds(r * COLS_W, COLS_W)
        pltpu.make_async_copy(table_h.at[idx_v.at[s_row]],
                              res_v.at[s_row], sems.at[r]).wait()
        pltpu.async_copy(res_v.at[s_row],
                         out_h.at[n_base + r, pl.ds(col0, COLS_W)], sem2)
    for r in range(ROWS_W):
        pltpu.make_async_copy(res_v.at[pl.ds(r * COLS_W, COLS_W)],
                              out_h.at[n_base + r, pl.ds(col0, COLS_W)],
                              sem2).wait()


@jax.jit
def _sc_gather(table, a, b, c):
    mesh = plsc.VectorSubcoreMesh(core_axis_name="c", subcore_axis_name="s")
    return pl.kernel(
        _sc_body,
        mesh=mesh,
        out_type=jax.ShapeDtypeStruct((N, BATCH), jnp.float32),
        scratch_types=[
            pltpu.VMEM((PER_W,), jnp.int32),
            pltpu.VMEM((PER_W,), jnp.int32),
            pltpu.VMEM((PER_W,), jnp.int32),
            pltpu.VMEM((PER_W,), jnp.int32),
            pltpu.VMEM((PER_W,), jnp.float32),
            pltpu.SemaphoreType.DMA((ROWS_W,)),
            pltpu.SemaphoreType.DMA,
        ],
    )(table, a, b, c)


def kernel(joint, a, b, c):
    table = jnp.transpose(joint, (1, 2, 3, 0)).reshape(-1)
    return _sc_gather(
        table,
        a.astype(jnp.int32),
        b.astype(jnp.int32),
        c.astype(jnp.int32),
    )
